# Initial kernel scaffold; baseline (speedup 1.0000x reference)
import jax
import jax.numpy as jnp
from jax.experimental import pallas as pl


def kernel(query_features, key_features, head_idx, lsh_proj_g0, lsh_proj_g1):
    raise NotImplementedError("write your pallas kernel here")


# speedup vs baseline: 40.5769x; 40.5769x over previous
import functools

import jax
import jax.numpy as jnp
from jax import lax
from jax.experimental import pallas as pl
from jax.experimental.pallas import tpu as pltpu

B, L, D = 2, 2048, 64
G = 32
NH = 4
BW = 4.0
NB = 64
K = 64
BQ = 256
NEG = float(jnp.finfo(jnp.float32).min)


def _pack_weights():
    d = lax.broadcasted_iota(jnp.int32, (G, 2), 0)
    c = lax.broadcasted_iota(jnp.int32, (G, 2), 1)
    p = jnp.left_shift(jnp.int32(1), lax.rem(d, 16)).astype(jnp.float32)
    return jnp.where(c == d // 16, p, 0.0)


def _body(qf_ref, kf_ref, p0_ref, p1_ref, cand_ref, vals_ref):
    qf = qf_ref[0]
    kf = kf_ref[0]
    projs = (p0_ref[...], p1_ref[...])

    W = _pack_weights()

    qcodes, kcodes, qhash, khash = [], [], [], []
    for g in range(2):
        qg = qf[:, g * G:(g + 1) * G]
        kg = kf[:, g * G:(g + 1) * G]
        qb = (qg > 0).astype(jnp.float32)
        kb = (kg > 0).astype(jnp.float32)
        qcodes.append(lax.dot_general(qb, W, (((1,), (0,)), ((), ())),
                                      preferred_element_type=jnp.float32))
        kcodes.append(lax.dot_general(W, kb, (((0,), (1,)), ((), ())),
                                      preferred_element_type=jnp.float32))
        qy = lax.dot_general(qg, projs[g], (((1,), (0,)), ((), ())),
                             preferred_element_type=jnp.float32)
        ky = lax.dot_general(projs[g], kg, (((0,), (1,)), ((), ())),
                             preferred_element_type=jnp.float32)
        qh = jnp.floor(qy / BW)
        kh = jnp.floor(ky / BW)
        qhash.append(qh - jnp.floor(qh / NB) * NB)
        khash.append(kh - jnp.floor(kh / NB) * NB)

    code_eq = []
    for g in range(2):
        eq = ((qcodes[g][:, 0:1] == kcodes[g][0:1, :]) &
              (qcodes[g][:, 1:2] == kcodes[g][1:2, :]))
        code_eq.append(eq)
    screen = code_eq[0] | code_eq[1]
    any_match = jnp.sum(screen.astype(jnp.int32)) > 0

    cand_ref[...] = jnp.full((1, BQ, K), -1, dtype=jnp.int32)
    vals_ref[...] = jnp.full((1, BQ, K), NEG, dtype=jnp.float32)

    @pl.when(any_match)
    def _heavy():
        full_mask = jnp.zeros((BQ, L), dtype=jnp.bool_)
        for g in range(2):
            lsh = jnp.zeros((BQ, L), dtype=jnp.bool_)
            for h in range(NH):
                lsh = lsh | (qhash[g][:, h:h + 1] == khash[g][h:h + 1, :])
            full_mask = full_mask | (code_eq[g] & lsh)
        scores = lax.dot_general(qf, kf, (((1,), (1,)), ((), ())),
                                 preferred_element_type=jnp.float32)
        masked = jnp.where(full_mask, scores, NEG)
        iota_k = lax.broadcasted_iota(jnp.int32, (BQ, L), 1)
        iota_c = lax.broadcasted_iota(jnp.int32, (BQ, K), 1)

        def step(j, carry):
            m_vals, out_v, out_i = carry
            mx = jnp.max(m_vals, axis=1, keepdims=True)
            idx = jnp.min(jnp.where(m_vals == mx, iota_k, L),
                          axis=1, keepdims=True)
            col = iota_c == j
            out_v = jnp.where(col, mx, out_v)
            out_i = jnp.where(col, jnp.where(mx > NEG, idx, -1), out_i)
            return jnp.where(iota_k == idx, NEG, m_vals), out_v, out_i

        _, out_v, out_i = lax.fori_loop(
            0, K, step,
            (masked,
             jnp.full((BQ, K), NEG, dtype=jnp.float32),
             jnp.full((BQ, K), -1, dtype=jnp.int32)))
        vals_ref[0] = out_v
        cand_ref[0] = out_i


@jax.jit
def _run(qf, kf, p0, p1):
    qt = L // BQ
    grid = (B, qt)
    return pl.pallas_call(
        _body,
        grid=grid,
        in_specs=[
            pl.BlockSpec((1, BQ, D), lambda b, t: (b, t, 0)),
            pl.BlockSpec((1, L, D), lambda b, t: (b, 0, 0)),
            pl.BlockSpec((G, NH), lambda b, t: (0, 0)),
            pl.BlockSpec((G, NH), lambda b, t: (0, 0)),
        ],
        out_specs=[
            pl.BlockSpec((1, BQ, K), lambda b, t: (b, t, 0)),
            pl.BlockSpec((1, BQ, K), lambda b, t: (b, t, 0)),
        ],
        out_shape=[
            jax.ShapeDtypeStruct((B, L, K), jnp.int32),
            jax.ShapeDtypeStruct((B, L, K), jnp.float32),
        ],
    )(qf, kf, p0, p1)


def kernel(query_features, key_features, head_idx, lsh_proj_g0, lsh_proj_g1):
    cand, vals = _run(query_features, key_features, lsh_proj_g0, lsh_proj_g1)
    return cand, vals
